# Initial kernel scaffold; baseline (speedup 1.0000x reference)
import jax
import jax.numpy as jnp
from jax.experimental import pallas as pl


def kernel(grad_key_cache, grad_value_cache, key_states, cos, sin, cache_position):
    raise NotImplementedError("write your pallas kernel here")


# speedup vs baseline: 2.3381x; 2.3381x over previous
import functools

import jax
import jax.numpy as jnp
from jax.experimental import pallas as pl
from jax.experimental.pallas import tpu as pltpu

_B, _H, _MAX, _NEW, _D = 8, 8, 4096, 16, 128
_HALF = _D // 2
_BF = jnp.bfloat16


def _body(pos_ref, kc_ref, vc_ref, ks_ref, cos_ref, sin_ref,
          gks_ref, gvs_ref, gcos_ref, gsin_ref, kco_ref, vco_ref,
          gk_ref, acc_cos_ref, acc_sin_ref, mask_ref):
    b = pl.program_id(0)
    h = pl.program_id(1)

    @pl.when((b == 0) & (h == 0))
    def _():
        ids = jax.lax.broadcasted_iota(jnp.int32, (_MAX, 1), 0)
        m = jnp.ones((_MAX, 1), jnp.float32)
        for j in range(_NEW):
            m = jnp.where(ids == pos_ref[j], 0.0, m)
        mask_ref[...] = m

    m = mask_ref[...]
    kco_ref[0, 0] = (kc_ref[0, 0] * m).astype(_BF)
    vco_ref[0, 0] = (vc_ref[0, 0] * m).astype(_BF)

    for j in range(_NEW):
        p = pos_ref[j]
        gk_ref[pl.ds(j, 1), :] = kc_ref[0, 0, pl.ds(p, 1), :]
        gvs_ref[0, 0, pl.ds(j, 1), :] = (
            vc_ref[0, 0, pl.ds(p, 1), :].astype(_BF))

    g = gk_ref[...]
    cosb = cos_ref[0]
    sinb = sin_ref[0]
    gcos_term = g * cosb
    gsin_term = g * sinb
    gk1 = gcos_term[:, :_HALF] + gsin_term[:, _HALF:]
    gk2 = gcos_term[:, _HALF:] - gsin_term[:, :_HALF]
    gks_ref[0, 0] = jnp.concatenate([gk1, gk2], axis=-1).astype(_BF)

    k = ks_ref[0, 0]
    krot = jnp.concatenate([-k[:, _HALF:], k[:, :_HALF]], axis=-1)
    pc = g * k
    ps = g * krot

    @pl.when(h == 0)
    def _():
        acc_cos_ref[...] = pc
        acc_sin_ref[...] = ps

    @pl.when(h != 0)
    def _():
        acc_cos_ref[...] += pc
        acc_sin_ref[...] += ps

    @pl.when(h == _H - 1)
    def _():
        gcos_ref[0] = acc_cos_ref[...].astype(_BF)
        gsin_ref[0] = acc_sin_ref[...].astype(_BF)


@jax.jit
def kernel(grad_key_cache, grad_value_cache, key_states, cos, sin,
           cache_position):
    grid = (_B, _H)
    cache_spec = pl.BlockSpec((1, 1, _MAX, _D), lambda b, h: (b, h, 0, 0))
    new_spec = pl.BlockSpec((1, 1, _NEW, _D), lambda b, h: (b, h, 0, 0))
    trig_spec = pl.BlockSpec((1, _NEW, _D), lambda b, h: (b, 0, 0))
    out = pl.pallas_call(
        _body,
        grid=grid,
        in_specs=[
            pl.BlockSpec(memory_space=pltpu.SMEM),
            cache_spec,
            cache_spec,
            new_spec,
            trig_spec,
            trig_spec,
        ],
        out_specs=[
            new_spec,
            new_spec,
            trig_spec,
            trig_spec,
            cache_spec,
            cache_spec,
        ],
        out_shape=[
            jax.ShapeDtypeStruct((_B, _H, _NEW, _D), _BF),
            jax.ShapeDtypeStruct((_B, _H, _NEW, _D), _BF),
            jax.ShapeDtypeStruct((_B, _NEW, _D), _BF),
            jax.ShapeDtypeStruct((_B, _NEW, _D), _BF),
            jax.ShapeDtypeStruct((_B, _H, _MAX, _D), _BF),
            jax.ShapeDtypeStruct((_B, _H, _MAX, _D), _BF),
        ],
        scratch_shapes=[
            pltpu.VMEM((_NEW, _D), jnp.float32),
            pltpu.VMEM((_NEW, _D), jnp.float32),
            pltpu.VMEM((_NEW, _D), jnp.float32),
            pltpu.VMEM((_MAX, 1), jnp.float32),
        ],
        compiler_params=pltpu.CompilerParams(
            dimension_semantics=("arbitrary", "arbitrary"),
        ),
    )(cache_position, grad_key_cache, grad_value_cache, key_states, cos, sin)
    gks, gvs, gcos, gsin, kco, vco = out
    return (gks, gvs, gcos, gsin, kco, vco)
